# baseline (speedup 1.0000x reference)
import functools
import numpy as np
import jax
import jax.numpy as jnp
from jax import lax
from jax.experimental import pallas as pl
from jax.experimental.pallas import tpu as pltpu
from jax.experimental.pallas import tpu_sc as plsc

_N = 64
_Q_FRACS = np.asarray([0.1, 0.2, 0.3, 0.4, 0.5, 0.6, 0.7, 0.8, 0.9], np.float32)
_IDX = np.round(_Q_FRACS * (_N - 1)).astype(np.int32)
_NQ = _IDX.shape[0]
_ROWS = 1024 * 512

_ROWS_SC = 262144
_ROWS_TC = _ROWS - _ROWS_SC
_SC_BASE = _ROWS_TC


_RB = 512
_NW = 32
_RPW = _ROWS_SC // _NW
_NBLK = _RPW // _RB


def _sort_desc(v):
    return plsc.sort_key_val(v, v, descending=True)[0]


def _sort64(a, b, c, d):
    a = lax.sort(a)
    b = _sort_desc(b)
    c = lax.sort(c)
    d = _sort_desc(d)
    lo, hi = jnp.minimum(a, b), jnp.maximum(a, b)
    a2, b2 = lax.sort(lo), lax.sort(hi)
    lo, hi = jnp.minimum(c, d), jnp.maximum(c, d)
    c2, d2 = _sort_desc(hi), _sort_desc(lo)
    l0, l1 = jnp.minimum(a2, c2), jnp.minimum(b2, d2)
    h0, h1 = jnp.maximum(a2, c2), jnp.maximum(b2, d2)
    s0 = lax.sort(jnp.minimum(l0, l1))
    s1 = lax.sort(jnp.maximum(l0, l1))
    s2 = lax.sort(jnp.minimum(h0, h1))
    s3 = lax.sort(jnp.maximum(h0, h1))
    return s0, s1, s2, s3


def _make_sc_kernel():
    mesh = plsc.VectorSubcoreMesh(core_axis_name="c", subcore_axis_name="s")

    @functools.partial(
        pl.kernel,
        mesh=mesh,
        out_type=jax.ShapeDtypeStruct((_ROWS_SC * _NQ,), jnp.float32),
        scratch_types=[
            pltpu.VMEM((_RB * _N,), jnp.float32),
            pltpu.VMEM((_RB * _NQ + 8,), jnp.float32),
        ],
        compiler_params=pltpu.CompilerParams(needs_layout_passes=False),
    )
    def k(x_hbm, out_hbm, x_v, o_v):
        wid = lax.axis_index("s") * 2 + lax.axis_index("c")
        lane = lax.iota(jnp.int32, 16)
        m0 = (lane == 6) | (lane == 13)
        m1 = (lane == 3) | (lane == 9)
        m2 = (lane == 0) | (lane == 6) | (lane == 12)
        m3 = (lane == 2) | (lane == 9)
        i0 = jnp.where(lane == 13, 1, 0)
        i1 = jnp.where(lane == 3, 2, 3)
        i2 = jnp.where(lane == 0, 4, jnp.where(lane == 6, 5, 6))
        i3 = jnp.where(lane == 2, 7, 8)

        def block_body(blk, carry):
            start = _SC_BASE + wid * _RPW + blk * _RB
            pltpu.sync_copy(x_hbm.at[pl.ds(start * _N, _RB * _N)], x_v)

            @plsc.parallel_loop(0, _RB, 1, unroll=4)
            def row_body(r):
                base = r * _N
                a = x_v[pl.ds(base, 16)]
                b = x_v[pl.ds(base + 16, 16)]
                c = x_v[pl.ds(base + 32, 16)]
                d = x_v[pl.ds(base + 48, 16)]
                s0, s1, s2, s3 = _sort64(a, b, c, d)
                r9 = r * _NQ
                plsc.store_scatter(o_v, [i0 + r9], s0, mask=m0)
                plsc.store_scatter(o_v, [i1 + r9], s1, mask=m1)
                plsc.store_scatter(o_v, [i2 + r9], s2, mask=m2)
                plsc.store_scatter(o_v, [i3 + r9], s3, mask=m3)

            pltpu.sync_copy(
                o_v.at[pl.ds(0, _RB * _NQ)],
                out_hbm.at[pl.ds((wid * _RPW + blk * _RB) * _NQ, _RB * _NQ)],
            )
            return carry

        lax.fori_loop(0, _NBLK, block_body, 0)

    return k


def _bitrev6(v):
    r = 0
    for b in range(6):
        r |= ((v >> b) & 1) << (5 - b)
    return r


_SEL = np.zeros((_NQ, _N), np.float32)
for _q, _r in enumerate(_IDX):
    _SEL[_q, _bitrev6(int(_r))] = 1.0

_SUB = 256
_NSUB = 4
_BLK = _SUB * _NSUB


def _sort_subtile(v, ri):
    for big_k in range(1, 7):
        kb = 1 << (5 - big_k) if big_k < 6 else 0
        for big_j in range(big_k - 1, -1, -1):
            sd = 1 << (5 - big_j)
            if sd >= 8:
                p = sd // 8
                g2 = 8 // (2 * p)
                v5 = v.reshape(g2, 2, p, 8, _SUB)
                lo, hi = v5[:, 0], v5[:, 1]
                mn, mx = jnp.minimum(lo, hi), jnp.maximum(lo, hi)
                if kb:
                    m = ((ri & kb) == 0).reshape(g2, 2, p, 8, _SUB)[:, 0]
                    lo2 = jnp.where(m, mn, mx)
                    hi2 = jnp.where(m, mx, mn)
                else:
                    lo2, hi2 = mn, mx
                v = jnp.concatenate([lo2[:, None], hi2[:, None]], axis=1)
                v = v.reshape(_N, _SUB)
            else:
                up = pltpu.roll(v, sd, 0)
                dn = pltpu.roll(v, _N - sd, 0)
                bit = (ri & sd) != 0
                partner = jnp.where(bit, up, dn)
                tm = jnp.logical_not(bit)
                if kb:
                    tm = tm == ((ri & kb) == 0)
                v = jnp.where(tm, jnp.minimum(v, partner), jnp.maximum(v, partner))
    return v


def _tc_body(sel_ref, x_ref, o_ref):
    ri = jax.lax.broadcasted_iota(jnp.int32, (_N, _SUB), 0)
    selt = sel_ref[...]
    for s in range(_NSUB):
        v = jnp.transpose(x_ref[pl.ds(s * _SUB, _SUB), :], (1, 0))
        v = _sort_subtile(v, ri)
        o_ref[pl.ds(s * _SUB, _SUB), :] = lax.dot_general(
            v, selt, (((0,), (0,)), ((), ())),
            preferred_element_type=jnp.float32,
            precision=jax.lax.Precision.HIGHEST)


def _tc_half(xv):
    grid = _ROWS_TC // _BLK
    return pl.pallas_call(
        _tc_body,
        grid=(grid,),
        in_specs=[
            pl.BlockSpec((_N, _NQ), lambda i: (0, 0)),
            pl.BlockSpec((_BLK, _N), lambda i: (i, 0)),
        ],
        out_specs=pl.BlockSpec((_BLK, _NQ), lambda i: (i, 0)),
        out_shape=jax.ShapeDtypeStruct((_ROWS_TC, _NQ), jnp.float32),
    )(jnp.asarray(_SEL.T.copy()), xv)


def kernel(x):
    b, t, n = x.shape
    xv = x.reshape(b * t, n)
    out_sc = _make_sc_kernel()(x.reshape(-1))
    out_tc = _tc_half(xv)
    y_sc = out_sc.reshape(_ROWS_SC, _NQ)
    return jnp.concatenate([out_tc, y_sc], axis=0).reshape(b, t, _NQ)


# speedup vs baseline: 1.2959x; 1.2959x over previous
import functools
import numpy as np
import jax
import jax.numpy as jnp
from jax import lax
from jax.experimental import pallas as pl
from jax.experimental.pallas import tpu as pltpu
from jax.experimental.pallas import tpu_sc as plsc

_N = 64
_Q_FRACS = np.asarray([0.1, 0.2, 0.3, 0.4, 0.5, 0.6, 0.7, 0.8, 0.9], np.float32)
_IDX = np.round(_Q_FRACS * (_N - 1)).astype(np.int32)
_NQ = _IDX.shape[0]
_ROWS = 1024 * 512

_ROWS_SC = _ROWS
_ROWS_TC = _ROWS - _ROWS_SC
_SC_BASE = _ROWS_TC


_RB = 512
_NW = 32
_RPW = _ROWS_SC // _NW
_NBLK = _RPW // _RB


def _sort_desc(v):
    return plsc.sort_key_val(v, v, descending=True)[0]


def _sort64(a, b, c, d):
    a = lax.sort(a)
    b = _sort_desc(b)
    c = lax.sort(c)
    d = _sort_desc(d)
    lo, hi = jnp.minimum(a, b), jnp.maximum(a, b)
    a2, b2 = lax.sort(lo), lax.sort(hi)
    lo, hi = jnp.minimum(c, d), jnp.maximum(c, d)
    c2, d2 = _sort_desc(hi), _sort_desc(lo)
    l0, l1 = jnp.minimum(a2, c2), jnp.minimum(b2, d2)
    h0, h1 = jnp.maximum(a2, c2), jnp.maximum(b2, d2)
    s0 = lax.sort(jnp.minimum(l0, l1))
    s1 = lax.sort(jnp.maximum(l0, l1))
    s2 = lax.sort(jnp.minimum(h0, h1))
    s3 = lax.sort(jnp.maximum(h0, h1))
    return s0, s1, s2, s3


def _make_sc_kernel():
    mesh = plsc.VectorSubcoreMesh(core_axis_name="c", subcore_axis_name="s")

    @functools.partial(
        pl.kernel,
        mesh=mesh,
        out_type=jax.ShapeDtypeStruct((_ROWS_SC * _NQ,), jnp.float32),
        scratch_types=[
            pltpu.VMEM((_RB * _N,), jnp.float32),
            pltpu.VMEM((_RB * _NQ + 8,), jnp.float32),
        ],
        compiler_params=pltpu.CompilerParams(needs_layout_passes=False),
    )
    def k(x_hbm, out_hbm, x_v, o_v):
        wid = lax.axis_index("s") * 2 + lax.axis_index("c")
        lane = lax.iota(jnp.int32, 16)
        m0 = (lane == 6) | (lane == 13)
        m1 = (lane == 3) | (lane == 9)
        m2 = (lane == 0) | (lane == 6) | (lane == 12)
        m3 = (lane == 2) | (lane == 9)
        i0 = jnp.where(lane == 13, 1, 0)
        i1 = jnp.where(lane == 3, 2, 3)
        i2 = jnp.where(lane == 0, 4, jnp.where(lane == 6, 5, 6))
        i3 = jnp.where(lane == 2, 7, 8)

        def block_body(blk, carry):
            start = _SC_BASE + wid * _RPW + blk * _RB
            pltpu.sync_copy(x_hbm.at[pl.ds(start * _N, _RB * _N)], x_v)

            @plsc.parallel_loop(0, _RB, 1, unroll=8)
            def row_body(r):
                base = r * _N
                a = x_v[pl.ds(base, 16)]
                b = x_v[pl.ds(base + 16, 16)]
                c = x_v[pl.ds(base + 32, 16)]
                d = x_v[pl.ds(base + 48, 16)]
                s0, s1, s2, s3 = _sort64(a, b, c, d)
                r9 = r * _NQ
                plsc.store_scatter(o_v, [i0 + r9], s0, mask=m0)
                plsc.store_scatter(o_v, [i1 + r9], s1, mask=m1)
                plsc.store_scatter(o_v, [i2 + r9], s2, mask=m2)
                plsc.store_scatter(o_v, [i3 + r9], s3, mask=m3)

            pltpu.sync_copy(
                o_v.at[pl.ds(0, _RB * _NQ)],
                out_hbm.at[pl.ds((wid * _RPW + blk * _RB) * _NQ, _RB * _NQ)],
            )
            return carry

        lax.fori_loop(0, _NBLK, block_body, 0)

    return k


def _bitrev6(v):
    r = 0
    for b in range(6):
        r |= ((v >> b) & 1) << (5 - b)
    return r


_SEL = np.zeros((_NQ, _N), np.float32)
for _q, _r in enumerate(_IDX):
    _SEL[_q, _bitrev6(int(_r))] = 1.0

_SUB = 256
_NSUB = 4
_BLK = _SUB * _NSUB


def _sort_subtile(v, ri):
    for big_k in range(1, 7):
        kb = 1 << (5 - big_k) if big_k < 6 else 0
        for big_j in range(big_k - 1, -1, -1):
            sd = 1 << (5 - big_j)
            if sd >= 8:
                p = sd // 8
                g2 = 8 // (2 * p)
                v5 = v.reshape(g2, 2, p, 8, _SUB)
                lo, hi = v5[:, 0], v5[:, 1]
                mn, mx = jnp.minimum(lo, hi), jnp.maximum(lo, hi)
                if kb:
                    m = ((ri & kb) == 0).reshape(g2, 2, p, 8, _SUB)[:, 0]
                    lo2 = jnp.where(m, mn, mx)
                    hi2 = jnp.where(m, mx, mn)
                else:
                    lo2, hi2 = mn, mx
                v = jnp.concatenate([lo2[:, None], hi2[:, None]], axis=1)
                v = v.reshape(_N, _SUB)
            else:
                up = pltpu.roll(v, sd, 0)
                dn = pltpu.roll(v, _N - sd, 0)
                bit = (ri & sd) != 0
                partner = jnp.where(bit, up, dn)
                tm = jnp.logical_not(bit)
                if kb:
                    tm = tm == ((ri & kb) == 0)
                v = jnp.where(tm, jnp.minimum(v, partner), jnp.maximum(v, partner))
    return v


def _tc_body(sel_ref, x_ref, o_ref):
    ri = jax.lax.broadcasted_iota(jnp.int32, (_N, _SUB), 0)
    selt = sel_ref[...]
    for s in range(_NSUB):
        v = jnp.transpose(x_ref[pl.ds(s * _SUB, _SUB), :], (1, 0))
        v = _sort_subtile(v, ri)
        o_ref[pl.ds(s * _SUB, _SUB), :] = lax.dot_general(
            v, selt, (((0,), (0,)), ((), ())),
            preferred_element_type=jnp.float32,
            precision=jax.lax.Precision.HIGHEST)


def _tc_half(xv):
    grid = _ROWS_TC // _BLK
    return pl.pallas_call(
        _tc_body,
        grid=(grid,),
        in_specs=[
            pl.BlockSpec((_N, _NQ), lambda i: (0, 0)),
            pl.BlockSpec((_BLK, _N), lambda i: (i, 0)),
        ],
        out_specs=pl.BlockSpec((_BLK, _NQ), lambda i: (i, 0)),
        out_shape=jax.ShapeDtypeStruct((_ROWS_TC, _NQ), jnp.float32),
    )(jnp.asarray(_SEL.T.copy()), xv)


def kernel(x):
    b, t, n = x.shape
    out_sc = _make_sc_kernel()(x.reshape(-1))
    if _ROWS_TC:
        out_tc = _tc_half(x.reshape(b * t, n))
        y_sc = out_sc.reshape(_ROWS_SC, _NQ)
        return jnp.concatenate([out_tc, y_sc], axis=0).reshape(b, t, _NQ)
    return out_sc.reshape(b, t, _NQ)


# speedup vs baseline: 1.4146x; 1.0916x over previous
import functools
import jax
import jax.numpy as jnp
from jax import lax
from jax.experimental import pallas as pl
from jax.experimental.pallas import tpu as pltpu
from jax.experimental.pallas import tpu_sc as plsc

_N = 64
_NQ = 9
_RB = 512
_NW = 32
_ROWS = 1024 * 512
_RPW = _ROWS // _NW
_NBLK = _RPW // _RB


def _sort_desc(v):
    return plsc.sort_key_val(v, v, descending=True)[0]


def _sort64(a, b, c, d):
    a = lax.sort(a)
    b = _sort_desc(b)
    c = lax.sort(c)
    d = _sort_desc(d)
    lo, hi = jnp.minimum(a, b), jnp.maximum(a, b)
    a2, b2 = lax.sort(lo), lax.sort(hi)
    lo, hi = jnp.minimum(c, d), jnp.maximum(c, d)
    c2, d2 = _sort_desc(hi), _sort_desc(lo)
    l0, l1 = jnp.minimum(a2, c2), jnp.minimum(b2, d2)
    h0, h1 = jnp.maximum(a2, c2), jnp.maximum(b2, d2)
    s0 = lax.sort(jnp.minimum(l0, l1))
    s1 = lax.sort(jnp.maximum(l0, l1))
    s2 = lax.sort(jnp.minimum(h0, h1))
    s3 = lax.sort(jnp.maximum(h0, h1))
    return s0, s1, s2, s3


def _make_kernel():
    mesh = plsc.VectorSubcoreMesh(core_axis_name="c", subcore_axis_name="s")

    @functools.partial(
        pl.kernel,
        mesh=mesh,
        out_type=jax.ShapeDtypeStruct((_ROWS * _NQ,), jnp.float32),
        scratch_types=[
            pltpu.VMEM((_RB * _N,), jnp.float32),
            pltpu.VMEM((_RB * _N,), jnp.float32),
            pltpu.VMEM((_RB * _NQ + 8,), jnp.float32),
            pltpu.SemaphoreType.DMA,
            pltpu.SemaphoreType.DMA,
        ],
        compiler_params=pltpu.CompilerParams(needs_layout_passes=False),
    )
    def k(x_hbm, out_hbm, x_v0, x_v1, o_v, sem0, sem1):
        wid = lax.axis_index("s") * 2 + lax.axis_index("c")
        base_row = wid * _RPW
        lane = lax.iota(jnp.int32, 16)
        m0 = (lane == 6) | (lane == 13)
        m1 = (lane == 3) | (lane == 9)
        m2 = (lane == 0) | (lane == 6) | (lane == 12)
        m3 = (lane == 2) | (lane == 9)
        i0 = jnp.where(lane == 13, 1, 0)
        i1 = jnp.where(lane == 3, 2, 3)
        i2 = jnp.where(lane == 0, 4, jnp.where(lane == 6, 5, 6))
        i3 = jnp.where(lane == 2, 7, 8)

        def in_copy(blk, buf, sem):
            start = base_row + blk * _RB
            return pltpu.make_async_copy(
                x_hbm.at[pl.ds(start * _N, _RB * _N)], buf, sem)

        def process(blk, buf):
            @plsc.parallel_loop(0, _RB, 1, unroll=4)
            def row_body(r):
                base = r * _N
                a = buf[pl.ds(base, 16)]
                b = buf[pl.ds(base + 16, 16)]
                c = buf[pl.ds(base + 32, 16)]
                d = buf[pl.ds(base + 48, 16)]
                s0, s1, s2, s3 = _sort64(a, b, c, d)
                r9 = r * _NQ
                plsc.store_scatter(o_v, [i0 + r9], s0, mask=m0)
                plsc.store_scatter(o_v, [i1 + r9], s1, mask=m1)
                plsc.store_scatter(o_v, [i2 + r9], s2, mask=m2)
                plsc.store_scatter(o_v, [i3 + r9], s3, mask=m3)

            start = base_row + blk * _RB
            pltpu.sync_copy(
                o_v.at[pl.ds(0, _RB * _NQ)],
                out_hbm.at[pl.ds(start * _NQ, _RB * _NQ)],
            )

        in_copy(0, x_v0, sem0).start()
        in_copy(1, x_v1, sem1).start()

        def pair_body(p, carry):
            blk = 2 * p
            in_copy(blk, x_v0, sem0).wait()
            process(blk, x_v0)

            @pl.when(blk + 2 < _NBLK)
            def _():
                in_copy(blk + 2, x_v0, sem0).start()

            in_copy(blk + 1, x_v1, sem1).wait()
            process(blk + 1, x_v1)

            @pl.when(blk + 3 < _NBLK)
            def _():
                in_copy(blk + 3, x_v1, sem1).start()

            return carry

        lax.fori_loop(0, _NBLK // 2, pair_body, 0)

    return k


def kernel(x):
    b, t, n = x.shape
    out = _make_kernel()(x.reshape(-1))
    return out.reshape(b, t, _NQ)
